# Initial kernel scaffold; baseline (speedup 1.0000x reference)
import jax
import jax.numpy as jnp
from jax.experimental import pallas as pl


def kernel(x, W_feat, b_feat, W_lat, b_lat, W_out, b_out):
    raise NotImplementedError("write your pallas kernel here")


# speedup vs baseline: 5.6955x; 5.6955x over previous
import functools

import jax
import jax.numpy as jnp
from jax import lax
from jax.experimental import pallas as pl
from jax.experimental.pallas import tpu as pltpu
from jax.experimental.pallas import tpu_sc as plsc

N = 10000
NPAD = 10240
DIM = 256
LDIM = 4
K = 16
TILE = 128
GRID = NPAD // TILE

NC = 2
NW = 32
NODES_PER_W = NPAD // NW
CN = 8
NCHUNK = NODES_PER_W // CN
DC = DIM // 16


def _featlat_body(x_ref, wf_ref, bf_ref, wl_ref, bl_ref, feat_ref, s_ref):
    xt = x_ref[...]
    feat_ref[...] = jnp.tanh(
        jnp.dot(xt, wf_ref[...], preferred_element_type=jnp.float32) + bf_ref[...])
    s_ref[...] = jnp.tanh(
        jnp.dot(xt, wl_ref[...], preferred_element_type=jnp.float32) + bl_ref[...])


def _topk_body(sT_ref, srow_ref, idx_ref, w_ref):
    i = pl.program_id(0)
    sT = sT_ref[...]
    srow = srow_ref[...]
    colsq = jnp.sum(sT * sT, axis=0, keepdims=True)
    rowsq = jnp.sum(srow * srow, axis=1, keepdims=True)
    sTb = sT.astype(jnp.bfloat16).astype(jnp.float32)
    srb = srow.astype(jnp.bfloat16).astype(jnp.float32)
    cross = srb[:, 0:1] * sTb[0:1, :]
    for j in range(1, LDIM):
        cross = cross + srb[:, j:j + 1] * sTb[j:j + 1, :]
    d2 = jnp.maximum(rowsq + colsq - 2.0 * cross, 0.0)
    colid = lax.broadcasted_iota(jnp.int32, (TILE, NPAD), 1)
    rowid = lax.broadcasted_iota(jnp.int32, (TILE, NPAD), 0) + i * TILE
    run = jnp.where((colid == rowid) | (colid >= N), jnp.inf, d2)
    idxs = []
    vals = []
    for _ in range(K):
        m = jnp.min(run, axis=1, keepdims=True)
        cand = jnp.where(run == m, colid, NPAD)
        a = jnp.min(cand, axis=1, keepdims=True)
        idxs.append(a)
        vals.append(m)
        run = jnp.where(colid == a, jnp.inf, run)
    idx_ref[...] = jnp.concatenate(idxs, axis=1)
    w_ref[...] = jnp.exp(-(jnp.concatenate(vals, axis=1) + 1e-12))


def _out_body(feat_ref, agg_ref, wo1_ref, wo2_ref, bo_ref, o_ref):
    o_ref[...] = (
        jnp.dot(feat_ref[...], wo1_ref[...], preferred_element_type=jnp.float32)
        + jnp.dot(agg_ref[...], wo2_ref[...], preferred_element_type=jnp.float32)
        + bo_ref[...])


@functools.cache
def _make_sc_agg():
    mesh = plsc.VectorSubcoreMesh(core_axis_name="c", subcore_axis_name="s")

    @functools.partial(
        pl.kernel,
        mesh=mesh,
        out_type=jax.ShapeDtypeStruct((NPAD, DIM), jnp.float32),
        scratch_types=[
            pltpu.VMEM((CN * K,), jnp.int32),
            pltpu.VMEM((CN * K,), jnp.float32),
            pltpu.VMEM((CN * K, DIM), jnp.float32),
            pltpu.VMEM((CN, DIM), jnp.float32),
            pltpu.SemaphoreType.DMA,
        ],
    )
    def _sc_agg(feat_hbm, idx_hbm, w_hbm, out_hbm, idx_v, w_v, rows_v, acc_v, sem):
        wid = lax.axis_index("s") * NC + lax.axis_index("c")

        def chunk_body(ci, carry):
            node_base = wid * NODES_PER_W + ci * CN
            edge_base = node_base * K
            pltpu.sync_copy(idx_hbm.at[pl.ds(edge_base, CN * K)], idx_v)
            pltpu.sync_copy(w_hbm.at[pl.ds(edge_base, CN * K)], w_v)
            pltpu.async_copy(feat_hbm.at[idx_v], rows_v, sem).wait()

            def node_body(n, carry2):
                wvec = w_v[pl.ds(n * K, K)]
                acc = [jnp.zeros((16,), jnp.float32) for _ in range(DC)]
                for kk in range(K):
                    wk = wvec[kk]
                    for c in range(DC):
                        acc[c] = acc[c] + wk * rows_v[n * K + kk, pl.ds(c * 16, 16)]
                for c in range(DC):
                    acc_v[n, pl.ds(c * 16, 16)] = acc[c]
                return carry2

            lax.fori_loop(0, CN, node_body, 0)
            pltpu.sync_copy(acc_v, out_hbm.at[pl.ds(node_base, CN)])
            return carry

        lax.fori_loop(0, NCHUNK, chunk_body, 0)

    return _sc_agg


def kernel(x, W_feat, b_feat, W_lat, b_lat, W_out, b_out):
    x_pad = jnp.pad(x, ((0, NPAD - N), (0, 0)))
    feat, s = pl.pallas_call(
        _featlat_body,
        grid=(GRID,),
        in_specs=[
            pl.BlockSpec((TILE, DIM), lambda i: (i, 0)),
            pl.BlockSpec((DIM, DIM), lambda i: (0, 0)),
            pl.BlockSpec((1, DIM), lambda i: (0, 0)),
            pl.BlockSpec((DIM, LDIM), lambda i: (0, 0)),
            pl.BlockSpec((1, LDIM), lambda i: (0, 0)),
        ],
        out_specs=[
            pl.BlockSpec((TILE, DIM), lambda i: (i, 0)),
            pl.BlockSpec((TILE, LDIM), lambda i: (i, 0)),
        ],
        out_shape=[
            jax.ShapeDtypeStruct((NPAD, DIM), jnp.float32),
            jax.ShapeDtypeStruct((NPAD, LDIM), jnp.float32),
        ],
    )(x_pad, W_feat, b_feat.reshape(1, DIM), W_lat, b_lat.reshape(1, LDIM))

    idx, w = pl.pallas_call(
        _topk_body,
        grid=(GRID,),
        in_specs=[
            pl.BlockSpec((LDIM, NPAD), lambda i: (0, 0)),
            pl.BlockSpec((TILE, LDIM), lambda i: (i, 0)),
        ],
        out_specs=[
            pl.BlockSpec((TILE, K), lambda i: (i, 0)),
            pl.BlockSpec((TILE, K), lambda i: (i, 0)),
        ],
        out_shape=[
            jax.ShapeDtypeStruct((NPAD, K), jnp.int32),
            jax.ShapeDtypeStruct((NPAD, K), jnp.float32),
        ],
    )(s.T, s)

    agg = _make_sc_agg()(feat, idx.reshape(-1), w.reshape(-1))

    out = pl.pallas_call(
        _out_body,
        grid=(GRID,),
        in_specs=[
            pl.BlockSpec((TILE, DIM), lambda i: (i, 0)),
            pl.BlockSpec((TILE, DIM), lambda i: (i, 0)),
            pl.BlockSpec((DIM, DIM), lambda i: (0, 0)),
            pl.BlockSpec((DIM, DIM), lambda i: (0, 0)),
            pl.BlockSpec((1, DIM), lambda i: (0, 0)),
        ],
        out_specs=pl.BlockSpec((TILE, DIM), lambda i: (i, 0)),
        out_shape=jax.ShapeDtypeStruct((NPAD, DIM), jnp.float32),
    )(feat, agg, W_out[:DIM], W_out[DIM:], b_out.reshape(1, DIM))
    return out[:N]
